# baseline (speedup 1.0000x reference)
import functools

import numpy as np
import jax
import jax.numpy as jnp
from jax import lax
from jax.experimental import pallas as pl
from jax.experimental.pallas import tpu as pltpu
from jax.experimental.pallas import tpu_sc as plsc

_IN_F = 8192
_OUT_F = 16384
_FAN = 327
_PAD_FAN = 336
_SEED = 42

_NC = 1
_NS = 16
_NW = _NC * _NS
_ROWS_PER_TILE = _IN_F // _NW
_CHUNK = 64
_NCHUNK = _ROWS_PER_TILE // _CHUNK
_VPR = _PAD_FAN // 16
_ACC_ROWS = 130


def _hash_targets_row_exact(src):
    seen = set()
    out = []
    conn = 0
    base = src * 2654435761 + _SEED
    while len(out) < _FAN:
        h = base + conn * 2246822519
        h = (h >> 16 ^ h) * 73244475
        h = (h >> 16 ^ h) * 73244475
        h = h >> 16 ^ h
        t = h % _OUT_F
        if t not in seen:
            seen.add(t)
            out.append(t)
        conn += 1
    return np.asarray(out, dtype=np.int32)


@functools.lru_cache(maxsize=1)
def _targets_padded():
    C = 512
    src = np.arange(_IN_F, dtype=np.uint64)[:, None]
    conn = np.arange(C, dtype=np.uint64)[None, :]
    h = src * np.uint64(2654435761) + np.uint64(_SEED) + conn * np.uint64(2246822519)
    h = ((h >> np.uint64(16)) ^ h) * np.uint64(73244475)
    h = ((h >> np.uint64(16)) ^ h) * np.uint64(73244475)
    h = (h >> np.uint64(16)) ^ h
    t = (h % np.uint64(_OUT_F)).astype(np.int32)
    T = np.empty((_IN_F, _FAN), dtype=np.int32)
    for r in range(_IN_F):
        row = t[r]
        _, first_idx = np.unique(row, return_index=True)
        if first_idx.size >= _FAN:
            first_idx.sort()
            T[r] = row[first_idx[:_FAN]]
        else:
            T[r] = _hash_targets_row_exact(r)
    ext = np.concatenate(
        [T[:, :320], T[:, 311:327], np.zeros((_IN_F, 16), np.int32)], axis=1)
    out = np.empty((_IN_F, 352), dtype=np.int16)
    for p in range(11):
        out[:, 32 * p: 32 * p + 32: 2] = ext[:, 32 * p: 32 * p + 16]
        out[:, 32 * p + 1: 32 * p + 32: 2] = ext[:, 32 * p + 16: 32 * p + 32]
    return np.ascontiguousarray(out).view(np.int32)


@functools.lru_cache(maxsize=1)
def _make_sc_kernel():
    mesh = plsc.VectorSubcoreMesh(
        core_axis_name="c", subcore_axis_name="s", num_cores=_NC)

    @functools.partial(
        pl.kernel,
        mesh=mesh,
        out_type=jax.ShapeDtypeStruct((128, 128), jnp.float32),
        compiler_params=pltpu.CompilerParams(needs_layout_passes=False),
        scratch_types=[
            pltpu.VMEM((_ROWS_PER_TILE + 16,), jnp.float32),
            pltpu.VMEM((2, _CHUNK, _FAN), jnp.float32),
            pltpu.VMEM((2, _CHUNK, 176), jnp.int32),
            pltpu.VMEM((128, 128), jnp.float32),
            pltpu.VMEM((128,), jnp.int32),
            pltpu.VMEM_SHARED((128, 128), jnp.float32),
            pltpu.SemaphoreType.DMA,
            pltpu.SemaphoreType.DMA,
            pltpu.SemaphoreType.DMA,
            pltpu.SemaphoreType.DMA,
            pltpu.SemaphoreType.DMA,
        ],
    )
    def sc_scatter(x_hbm, w_hbm, t_hbm, out_hbm,
                   x_v, w_v, t_v, acc_v, idx_v, shared,
                   wsem0, wsem1, tsem0, tsem1, xsem):
        sid = lax.axis_index("s")
        wid = sid
        base_row = wid * _ROWS_PER_TILE
        wsems = (wsem0, wsem1)
        tsems = (tsem0, tsem1)

        def start_chunk(k):
            b = k % 2
            row0 = base_row + k * _CHUNK
            return (
                pltpu.async_copy(w_hbm.at[pl.ds(row0, _CHUNK)], w_v.at[b], wsems[b]),
                pltpu.async_copy(t_hbm.at[pl.ds(row0, _CHUNK)], t_v.at[b], tsems[b]),
            )

        pending = {0: start_chunk(0)}
        x_copy = pltpu.async_copy(x_hbm.at[pl.ds(base_row, _ROWS_PER_TILE)],
                                  x_v.at[pl.ds(0, _ROWS_PER_TILE)], xsem)

        zeros16 = jnp.zeros((16,), jnp.float32)

        def zero_body(i, carry):
            for j in range(8):
                acc_v[i, pl.ds(j * 16, 16)] = zeros16
            return carry
        lax.fori_loop(0, 128, zero_body, 0)

        tail_mask = lax.broadcasted_iota(jnp.int32, (16,), 0) >= 9
        x_copy.wait()

        for k in range(_NCHUNK):
            if k + 1 < _NCHUNK:
                pending[k + 1] = start_chunk(k + 1)
            for h in pending.pop(k):
                h.wait()
            b = k % 2
            xoff = k * _CHUNK

            @plsc.parallel_loop(0, _CHUNK, unroll=2)
            def row_body(r):
                xr = jnp.broadcast_to(x_v[pl.ds(xoff + r, 16)][0], (16,))
                for p in range(10):
                    tt = plsc.bitcast(t_v[b, r, pl.ds(16 * p, 16)], jnp.int16)
                    ta, tb = plsc.unpack(
                        tt, format=plsc.PackFormat.INTERLEAVED,
                        preferred_element_type=jnp.int32)
                    wa = w_v[b, r, pl.ds(32 * p, 16)]
                    wb = w_v[b, r, pl.ds(32 * p + 16, 16)]
                    plsc.addupdate_scatter(
                        acc_v, [lax.shift_right_logical(ta, 7), ta & 127],
                        xr * wa)
                    plsc.addupdate_scatter(
                        acc_v, [lax.shift_right_logical(tb, 7), tb & 127],
                        xr * wb)
                tt = plsc.bitcast(t_v[b, r, pl.ds(160, 16)], jnp.int16)
                ta, _ = plsc.unpack(
                    tt, format=plsc.PackFormat.INTERLEAVED,
                    preferred_element_type=jnp.int32)
                wv = w_v[b, r, pl.ds(_FAN - 16, 16)]
                plsc.addupdate_scatter(
                    acc_v, [lax.shift_right_logical(ta, 7), ta & 127],
                    xr * wv, mask=tail_mask)

        lane16 = lax.broadcasted_iota(jnp.int32, (16,), 0)
        for j in range(8):
            idx_v[pl.ds(j * 16, 16)] = lane16 + (j * 16)

        @pl.when(sid == 0)
        def _():
            pltpu.sync_copy(acc_v, shared)
        plsc.subcore_barrier()

        @pl.when(sid != 0)
        def _():
            pltpu.sync_copy(acc_v, shared.at[idx_v], add=True)
        plsc.subcore_barrier()

        @pl.when(sid == 0)
        def _():
            pltpu.sync_copy(shared, out_hbm)

    return sc_scatter


def kernel(x, W):
    T = jnp.asarray(_targets_padded())
    parts = _make_sc_kernel()(x, W, T)
    return parts.reshape(_OUT_F)


# speedup vs baseline: 1.1748x; 1.1748x over previous
import functools

import numpy as np
import jax
import jax.numpy as jnp
from jax import lax
from jax.experimental import pallas as pl
from jax.experimental.pallas import tpu as pltpu
from jax.experimental.pallas import tpu_sc as plsc

_IN_F = 8192
_OUT_F = 16384
_FAN = 327
_PAD_FAN = 336
_SEED = 42

_NC = 2
_NS = 16
_NW = _NC * _NS
_ROWS_PER_TILE = _IN_F // _NW
_CHUNK = 32
_NCHUNK = _ROWS_PER_TILE // _CHUNK
_VPR = _PAD_FAN // 16
_ACC_ROWS = 130


def _hash_targets_row_exact(src):
    seen = set()
    out = []
    conn = 0
    base = src * 2654435761 + _SEED
    while len(out) < _FAN:
        h = base + conn * 2246822519
        h = (h >> 16 ^ h) * 73244475
        h = (h >> 16 ^ h) * 73244475
        h = h >> 16 ^ h
        t = h % _OUT_F
        if t not in seen:
            seen.add(t)
            out.append(t)
        conn += 1
    return np.asarray(out, dtype=np.int32)


@functools.lru_cache(maxsize=1)
def _targets_padded():
    C = 512
    src = np.arange(_IN_F, dtype=np.uint64)[:, None]
    conn = np.arange(C, dtype=np.uint64)[None, :]
    h = src * np.uint64(2654435761) + np.uint64(_SEED) + conn * np.uint64(2246822519)
    h = ((h >> np.uint64(16)) ^ h) * np.uint64(73244475)
    h = ((h >> np.uint64(16)) ^ h) * np.uint64(73244475)
    h = (h >> np.uint64(16)) ^ h
    t = (h % np.uint64(_OUT_F)).astype(np.int32)
    T = np.empty((_IN_F, _FAN), dtype=np.int32)
    for r in range(_IN_F):
        row = t[r]
        _, first_idx = np.unique(row, return_index=True)
        if first_idx.size >= _FAN:
            first_idx.sort()
            T[r] = row[first_idx[:_FAN]]
        else:
            T[r] = _hash_targets_row_exact(r)
    ext = np.concatenate(
        [T[:, :320], T[:, 311:327], np.zeros((_IN_F, 16), np.int32)], axis=1)
    out = np.empty((_IN_F, 352), dtype=np.int16)
    for p in range(11):
        out[:, 32 * p: 32 * p + 32: 2] = ext[:, 32 * p: 32 * p + 16]
        out[:, 32 * p + 1: 32 * p + 32: 2] = ext[:, 32 * p + 16: 32 * p + 32]
    return np.ascontiguousarray(out).view(np.int32)


@functools.lru_cache(maxsize=1)
def _make_sc_kernel():
    mesh = plsc.VectorSubcoreMesh(core_axis_name="c", subcore_axis_name="s")

    @functools.partial(
        pl.kernel,
        mesh=mesh,
        out_type=(jax.ShapeDtypeStruct((128, 128), jnp.float32),
                  jax.ShapeDtypeStruct((128, 128), jnp.float32)),
        compiler_params=pltpu.CompilerParams(needs_layout_passes=False),
        scratch_types=[
            pltpu.VMEM((_ROWS_PER_TILE + 16,), jnp.float32),
            pltpu.VMEM((2, _CHUNK, _FAN), jnp.float32),
            pltpu.VMEM((2, _CHUNK, 176), jnp.int32),
            pltpu.VMEM((128, 128), jnp.float32),
            pltpu.VMEM((128,), jnp.int32),
            pltpu.VMEM_SHARED((128, 128), jnp.float32),
            pltpu.SemaphoreType.DMA,
            pltpu.SemaphoreType.DMA,
            pltpu.SemaphoreType.DMA,
            pltpu.SemaphoreType.DMA,
            pltpu.SemaphoreType.DMA,
        ],
    )
    def sc_scatter(x_hbm, w_hbm, t_hbm, out0_hbm, out1_hbm,
                   x_v, w_v, t_v, acc_v, idx_v, shared,
                   wsem0, wsem1, tsem0, tsem1, xsem):
        cid = lax.axis_index("c")
        sid = lax.axis_index("s")
        wid = sid * _NC + cid
        base_row = wid * _ROWS_PER_TILE
        wsems = (wsem0, wsem1)
        tsems = (tsem0, tsem1)

        def start_chunk(k):
            b = k % 2
            row0 = base_row + k * _CHUNK
            return (
                pltpu.async_copy(w_hbm.at[pl.ds(row0, _CHUNK)], w_v.at[b], wsems[b]),
                pltpu.async_copy(t_hbm.at[pl.ds(row0, _CHUNK)], t_v.at[b], tsems[b]),
            )

        pending = {0: start_chunk(0)}
        x_copy = pltpu.async_copy(x_hbm.at[pl.ds(base_row, _ROWS_PER_TILE)],
                                  x_v.at[pl.ds(0, _ROWS_PER_TILE)], xsem)

        zeros16 = jnp.zeros((16,), jnp.float32)

        def zero_body(i, carry):
            for j in range(8):
                acc_v[i, pl.ds(j * 16, 16)] = zeros16
            return carry
        lax.fori_loop(0, 128, zero_body, 0)

        tail_mask = lax.broadcasted_iota(jnp.int32, (16,), 0) >= 9
        x_copy.wait()

        for k in range(_NCHUNK):
            if k + 1 < _NCHUNK:
                pending[k + 1] = start_chunk(k + 1)
            for h in pending.pop(k):
                h.wait()
            b = k % 2
            xoff = k * _CHUNK

            @plsc.parallel_loop(0, _CHUNK, unroll=2)
            def row_body(r):
                xr = jnp.broadcast_to(x_v[pl.ds(xoff + r, 16)][0], (16,))
                for p in range(10):
                    tt = plsc.bitcast(t_v[b, r, pl.ds(16 * p, 16)], jnp.int16)
                    ta, tb = plsc.unpack(
                        tt, format=plsc.PackFormat.INTERLEAVED,
                        preferred_element_type=jnp.int32)
                    wa = w_v[b, r, pl.ds(32 * p, 16)]
                    wb = w_v[b, r, pl.ds(32 * p + 16, 16)]
                    plsc.addupdate_scatter(
                        acc_v, [lax.shift_right_logical(ta, 7), ta & 127],
                        xr * wa)
                    plsc.addupdate_scatter(
                        acc_v, [lax.shift_right_logical(tb, 7), tb & 127],
                        xr * wb)
                tt = plsc.bitcast(t_v[b, r, pl.ds(160, 16)], jnp.int16)
                ta, _ = plsc.unpack(
                    tt, format=plsc.PackFormat.INTERLEAVED,
                    preferred_element_type=jnp.int32)
                wv = w_v[b, r, pl.ds(_FAN - 16, 16)]
                plsc.addupdate_scatter(
                    acc_v, [lax.shift_right_logical(ta, 7), ta & 127],
                    xr * wv, mask=tail_mask)

        lane16 = lax.broadcasted_iota(jnp.int32, (16,), 0)
        for j in range(8):
            idx_v[pl.ds(j * 16, 16)] = lane16 + (j * 16)

        @pl.when(sid == 0)
        def _():
            pltpu.sync_copy(acc_v, shared)
        plsc.subcore_barrier()

        @pl.when(sid != 0)
        def _():
            pltpu.sync_copy(acc_v, shared.at[idx_v], add=True)
        plsc.subcore_barrier()

        @pl.when(jnp.logical_and(sid == 0, cid == 0))
        def _():
            pltpu.sync_copy(shared, out0_hbm)

        @pl.when(jnp.logical_and(sid == 0, cid == 1))
        def _():
            pltpu.sync_copy(shared, out1_hbm)

    return sc_scatter


def _tc_add(a_ref, b_ref, o_ref):
    o_ref[...] = a_ref[...] + b_ref[...]


def kernel(x, W):
    T = jnp.asarray(_targets_padded())
    p0, p1 = _make_sc_kernel()(x, W, T)
    out = pl.pallas_call(
        _tc_add,
        out_shape=jax.ShapeDtypeStruct((_OUT_F,), jnp.float32),
    )(p0.reshape(_OUT_F), p1.reshape(_OUT_F))
    return out


# speedup vs baseline: 1.2181x; 1.0368x over previous
import functools

import numpy as np
import jax
import jax.numpy as jnp
from jax import lax
from jax.experimental import pallas as pl
from jax.experimental.pallas import tpu as pltpu
from jax.experimental.pallas import tpu_sc as plsc

_IN_F = 8192
_OUT_F = 16384
_FAN = 327
_SEED = 42

_NC = 2
_NS = 16
_NW = _NC * _NS
_ROWS_PER_TILE = _IN_F // _NW
_CHUNK = 64
_NCHUNK = _ROWS_PER_TILE // _CHUNK


def _hash_targets_row_exact(src):
    seen = set()
    out = []
    conn = 0
    base = src * 2654435761 + _SEED
    while len(out) < _FAN:
        h = base + conn * 2246822519
        h = (h >> 16 ^ h) * 73244475
        h = (h >> 16 ^ h) * 73244475
        h = h >> 16 ^ h
        t = h % _OUT_F
        if t not in seen:
            seen.add(t)
            out.append(t)
        conn += 1
    return np.asarray(out, dtype=np.int32)


@functools.lru_cache(maxsize=1)
def _targets_padded():
    C = 512
    src = np.arange(_IN_F, dtype=np.uint64)[:, None]
    conn = np.arange(C, dtype=np.uint64)[None, :]
    h = src * np.uint64(2654435761) + np.uint64(_SEED) + conn * np.uint64(2246822519)
    h = ((h >> np.uint64(16)) ^ h) * np.uint64(73244475)
    h = ((h >> np.uint64(16)) ^ h) * np.uint64(73244475)
    h = (h >> np.uint64(16)) ^ h
    t = (h % np.uint64(_OUT_F)).astype(np.int32)
    T = np.empty((_IN_F, _FAN), dtype=np.int32)
    for r in range(_IN_F):
        row = t[r]
        _, first_idx = np.unique(row, return_index=True)
        if first_idx.size >= _FAN:
            first_idx.sort()
            T[r] = row[first_idx[:_FAN]]
        else:
            T[r] = _hash_targets_row_exact(r)
    ext = np.concatenate(
        [T[:, :320], T[:, 311:327], np.zeros((_IN_F, 16), np.int32)], axis=1)
    out = np.empty((_IN_F, 352), dtype=np.int16)
    for p in range(11):
        out[:, 32 * p: 32 * p + 32: 2] = ext[:, 32 * p: 32 * p + 16]
        out[:, 32 * p + 1: 32 * p + 32: 2] = ext[:, 32 * p + 16: 32 * p + 32]
    return np.ascontiguousarray(out).view(np.int32)


@functools.lru_cache(maxsize=1)
def _make_sc_kernel():
    mesh = plsc.VectorSubcoreMesh(core_axis_name="c", subcore_axis_name="s")

    @functools.partial(
        pl.kernel,
        mesh=mesh,
        out_type=(jax.ShapeDtypeStruct((128, 128), jnp.float32),
                  jax.ShapeDtypeStruct((128, 128), jnp.float32)),
        compiler_params=pltpu.CompilerParams(needs_layout_passes=False),
        scratch_types=[
            pltpu.VMEM((_ROWS_PER_TILE + 16,), jnp.float32),
            pltpu.VMEM((2, _CHUNK, _FAN), jnp.float32),
            pltpu.VMEM((2, _CHUNK, 176), jnp.int32),
            pltpu.VMEM((128, 128), jnp.float32),
            pltpu.VMEM((128,), jnp.int32),
            pltpu.VMEM_SHARED((128, 128), jnp.float32),
            pltpu.SemaphoreType.DMA,
            pltpu.SemaphoreType.DMA,
            pltpu.SemaphoreType.DMA,
            pltpu.SemaphoreType.DMA,
            pltpu.SemaphoreType.DMA,
        ],
    )
    def sc_scatter(x_hbm, w_hbm, t_hbm, out0_hbm, out1_hbm,
                   x_v, w_v, t_v, acc_v, idx_v, shared,
                   wsem0, wsem1, tsem0, tsem1, xsem):
        cid = lax.axis_index("c")
        sid = lax.axis_index("s")
        wid = sid * _NC + cid
        base_row = wid * _ROWS_PER_TILE
        wsems = (wsem0, wsem1)
        tsems = (tsem0, tsem1)

        def start_chunk(k):
            b = k % 2
            row0 = base_row + k * _CHUNK
            return (
                pltpu.async_copy(w_hbm.at[pl.ds(row0, _CHUNK)], w_v.at[b], wsems[b]),
                pltpu.async_copy(t_hbm.at[pl.ds(row0, _CHUNK)], t_v.at[b], tsems[b]),
            )

        pending = {0: start_chunk(0)}
        x_copy = pltpu.async_copy(x_hbm.at[pl.ds(base_row, _ROWS_PER_TILE)],
                                  x_v.at[pl.ds(0, _ROWS_PER_TILE)], xsem)

        zeros16 = jnp.zeros((16,), jnp.float32)

        def zero_body(i, carry):
            for j in range(8):
                acc_v[i, pl.ds(j * 16, 16)] = zeros16
            return carry
        lax.fori_loop(0, 128, zero_body, 0)

        tail_mask = lax.broadcasted_iota(jnp.int32, (16,), 0) >= 9
        x_copy.wait()

        for k in range(_NCHUNK):
            if k + 1 < _NCHUNK:
                pending[k + 1] = start_chunk(k + 1)
            for h in pending.pop(k):
                h.wait()
            b = k % 2
            xoff = k * _CHUNK

            @plsc.parallel_loop(0, _CHUNK, unroll=2)
            def row_body(r):
                xr = jnp.broadcast_to(x_v[pl.ds(xoff + r, 16)][0], (16,))
                for p in range(10):
                    tt = plsc.bitcast(t_v[b, r, pl.ds(16 * p, 16)], jnp.int16)
                    ta, tb = plsc.unpack(
                        tt, format=plsc.PackFormat.INTERLEAVED,
                        preferred_element_type=jnp.int32)
                    wa = w_v[b, r, pl.ds(32 * p, 16)]
                    wb = w_v[b, r, pl.ds(32 * p + 16, 16)]
                    plsc.addupdate_scatter(
                        acc_v, [lax.shift_right_logical(ta, 7), ta & 127],
                        xr * wa)
                    plsc.addupdate_scatter(
                        acc_v, [lax.shift_right_logical(tb, 7), tb & 127],
                        xr * wb)
                tt = plsc.bitcast(t_v[b, r, pl.ds(160, 16)], jnp.int16)
                ta, _ = plsc.unpack(
                    tt, format=plsc.PackFormat.INTERLEAVED,
                    preferred_element_type=jnp.int32)
                wv = w_v[b, r, pl.ds(_FAN - 16, 16)]
                plsc.addupdate_scatter(
                    acc_v, [lax.shift_right_logical(ta, 7), ta & 127],
                    xr * wv, mask=tail_mask)

        lane16 = lax.broadcasted_iota(jnp.int32, (16,), 0)
        for j in range(8):
            idx_v[pl.ds(j * 16, 16)] = lane16 + (j * 16)

        @pl.when(sid == 0)
        def _():
            pltpu.sync_copy(acc_v, shared)
        plsc.subcore_barrier()

        @pl.when(sid != 0)
        def _():
            pltpu.sync_copy(acc_v, shared.at[idx_v], add=True)
        plsc.subcore_barrier()

        @pl.when(jnp.logical_and(sid == 0, cid == 0))
        def _():
            pltpu.sync_copy(shared, out0_hbm)

        @pl.when(jnp.logical_and(sid == 0, cid == 1))
        def _():
            pltpu.sync_copy(shared, out1_hbm)

    return sc_scatter


def _tc_add(a_ref, b_ref, o_ref):
    o_ref[...] = a_ref[...] + b_ref[...]


def kernel(x, W):
    T = jnp.asarray(_targets_padded())
    p0, p1 = _make_sc_kernel()(x, W, T)
    out = pl.pallas_call(
        _tc_add,
        out_shape=jax.ShapeDtypeStruct((_OUT_F,), jnp.float32),
    )(p0.reshape(_OUT_F), p1.reshape(_OUT_F))
    return out
